# Initial kernel scaffold; baseline (speedup 1.0000x reference)
import jax
import jax.numpy as jnp
from jax.experimental import pallas as pl


def kernel(x, Wq, bq, Wk, bk, Wv, bv, Wo, bo, gW1, gb1, gW2, gb2, gW3, gb3, nW1, nb1, nW2, nb2, nW3, nb3):
    raise NotImplementedError("write your pallas kernel here")


# speedup vs baseline: 1.1582x; 1.1582x over previous
import jax
import jax.numpy as jnp
from jax.experimental import pallas as pl
from jax.experimental.pallas import tpu as pltpu

D = 768
E = 64
TOP_K = 8
TOK = 32768
TILE = 1024


def _dot768(a, b_ref, acc_ref):
    acc_ref[...] = jnp.dot(a[:, 0:256], b_ref[0:256, :],
                           preferred_element_type=jnp.float32)
    acc_ref[...] = acc_ref[...] + jnp.dot(a[:, 256:512], b_ref[256:512, :],
                                          preferred_element_type=jnp.float32)
    acc_ref[...] = acc_ref[...] + jnp.dot(a[:, 512:768], b_ref[512:768, :],
                                          preferred_element_type=jnp.float32)
    return acc_ref[...]


def _router_kernel(x_ref, Wv_ref, bv_ref, Wo_ref, bo_ref,
                   nW1_ref, nb1_ref, nW2_ref, nb2_ref, nW3_ref, nb3_ref,
                   gW1_ref, gb1_ref, gW2_ref, gb2_ref, gW3_ref, gb3_ref,
                   idx_ref, w_ref, acc1_ref, acc2_ref, acc3_ref):
    xt = x_ref[...]
    ctx = _dot768(xt, Wv_ref, acc1_ref) + bv_ref[...]
    xs = jnp.dot(ctx, Wo_ref[...],
                 preferred_element_type=jnp.float32) + bo_ref[...]
    n = jax.nn.relu(_dot768(xs, nW1_ref, acc2_ref) + nb1_ref[...])
    n = jax.nn.relu(jnp.dot(n, nW2_ref[...],
                            preferred_element_type=jnp.float32) + nb2_ref[...])
    noisy = jax.nn.softplus(jnp.dot(n, nW3_ref[...],
                                    preferred_element_type=jnp.float32)
                            + nb3_ref[...])
    g = jax.nn.relu(_dot768(xs, gW1_ref, acc3_ref) + gb1_ref[...])
    g = jax.nn.relu(jnp.dot(g, gW2_ref[...],
                            preferred_element_type=jnp.float32) + gb2_ref[...])
    gating = jnp.dot(g, gW3_ref[...],
                     preferred_element_type=jnp.float32) + gb3_ref[...]
    mean = jnp.mean(noisy, axis=1, keepdims=True)
    var = jnp.sum((noisy - mean) ** 2, axis=1, keepdims=True) / (E - 1)
    std = jnp.sqrt(var)
    combined = gating + (noisy - mean) / (std + 1e-8)

    lanes = jax.lax.broadcasted_iota(jnp.int32, (TILE, E), 1)
    vals = combined
    top_v = []
    top_i = []
    for _ in range(TOP_K):
        m = jnp.max(vals, axis=1, keepdims=True)
        am = jnp.min(jnp.where(vals == m, lanes, E), axis=1, keepdims=True)
        top_v.append(m)
        top_i.append(am)
        vals = jnp.where(lanes == am, -jnp.inf, vals)
    tv = jnp.concatenate(top_v, axis=1)
    ti = jnp.concatenate(top_i, axis=1)
    e = jnp.exp(tv - tv[:, 0:1])
    w = e / jnp.sum(e, axis=1, keepdims=True)
    idx_ref[...] = ti
    w_ref[...] = w


@jax.jit
def kernel(x, Wq, bq, Wk, bk, Wv, bv, Wo, bo, gW1, gb1, gW2, gb2, gW3, gb3,
           nW1, nb1, nW2, nb2, nW3, nb3):
    x2 = x.reshape(TOK, D)
    grid = (TOK // TILE,)
    tok_spec = pl.BlockSpec((TILE, D), lambda i: (i, 0))
    out_spec = pl.BlockSpec((TILE, TOP_K), lambda i: (i, 0))

    def full(a):
        return pl.BlockSpec(a.shape, lambda i: (0,) * a.ndim)

    ops = [Wv, bv.reshape(1, -1), Wo, bo.reshape(1, -1),
           nW1, nb1.reshape(1, -1), nW2, nb2.reshape(1, -1),
           nW3, nb3.reshape(1, -1),
           gW1, gb1.reshape(1, -1), gW2, gb2.reshape(1, -1),
           gW3, gb3.reshape(1, -1)]

    idx, w = pl.pallas_call(
        _router_kernel,
        grid=grid,
        in_specs=[tok_spec] + [full(o) for o in ops],
        out_specs=[out_spec, out_spec],
        out_shape=[jax.ShapeDtypeStruct((TOK, TOP_K), jnp.int32),
                   jax.ShapeDtypeStruct((TOK, TOP_K), jnp.float32)],
        scratch_shapes=[pltpu.VMEM((TILE, 64), jnp.float32),
                        pltpu.VMEM((TILE, 64), jnp.float32),
                        pltpu.VMEM((TILE, 256), jnp.float32)],
    )(x2, *ops)
    return idx, w


# speedup vs baseline: 1.2638x; 1.0911x over previous
import jax
import jax.numpy as jnp
from jax.experimental import pallas as pl
from jax.experimental.pallas import tpu as pltpu

D = 768
E = 64
TOP_K = 8
TOK = 32768
TILE = 1024


def _dot768(a, b_ref, acc_ref):
    acc_ref[...] = jnp.dot(a[:, 0:256], b_ref[0:256, :],
                           preferred_element_type=jnp.float32)
    acc_ref[...] = acc_ref[...] + jnp.dot(a[:, 256:512], b_ref[256:512, :],
                                          preferred_element_type=jnp.float32)
    acc_ref[...] = acc_ref[...] + jnp.dot(a[:, 512:768], b_ref[512:768, :],
                                          preferred_element_type=jnp.float32)
    return acc_ref[...]


def _router_kernel(x_ref, Wv_ref, bv_ref, Wo_ref, bo_ref,
                   nW1_ref, nb1_ref, nW2_ref, nb2_ref, nW3_ref, nb3_ref,
                   gW1_ref, gb1_ref, gW2_ref, gb2_ref, gW3_ref, gb3_ref,
                   idx_ref, w_ref, acc1_ref, acc2_ref, acc3_ref):
    xt = x_ref[...]
    ctx = _dot768(xt, Wv_ref, acc1_ref) + bv_ref[...]
    xs = jnp.dot(ctx, Wo_ref[...],
                 preferred_element_type=jnp.float32) + bo_ref[...]
    n = jax.nn.relu(_dot768(xs, nW1_ref, acc2_ref) + nb1_ref[...])
    n = jax.nn.relu(jnp.dot(n, nW2_ref[...],
                            preferred_element_type=jnp.float32) + nb2_ref[...])
    noisy = jax.nn.softplus(jnp.dot(n, nW3_ref[...],
                                    preferred_element_type=jnp.float32)
                            + nb3_ref[...])
    g = jax.nn.relu(_dot768(xs, gW1_ref, acc3_ref) + gb1_ref[...])
    g = jax.nn.relu(jnp.dot(g, gW2_ref[...],
                            preferred_element_type=jnp.float32) + gb2_ref[...])
    gating = jnp.dot(g, gW3_ref[...],
                     preferred_element_type=jnp.float32) + gb3_ref[...]
    mean = jnp.mean(noisy, axis=1, keepdims=True)
    var = jnp.sum((noisy - mean) ** 2, axis=1, keepdims=True) / (E - 1)
    std = jnp.sqrt(var)
    combined = gating + (noisy - mean) / (std + 1e-8)

    lanes = jax.lax.broadcasted_iota(jnp.int32, (TILE, E), 1).astype(jnp.float32)
    vals = combined
    top_v = []
    top_i = []
    for _ in range(TOP_K):
        m = jnp.max(vals, axis=1, keepdims=True)
        am = jnp.min(jnp.where(vals == m, lanes, jnp.float32(E)), axis=1,
                     keepdims=True)
        top_v.append(m)
        top_i.append(am)
        vals = jnp.where(lanes == am, -jnp.inf, vals)
    tv = jnp.concatenate(top_v, axis=1)
    ti = jnp.concatenate(top_i, axis=1)
    e = jnp.exp(tv - tv[:, 0:1])
    w = e / jnp.sum(e, axis=1, keepdims=True)
    idx_ref[...] = ti.astype(jnp.int32)
    w_ref[...] = w


@jax.jit
def kernel(x, Wq, bq, Wk, bk, Wv, bv, Wo, bo, gW1, gb1, gW2, gb2, gW3, gb3,
           nW1, nb1, nW2, nb2, nW3, nb3):
    x2 = x.reshape(TOK, D)
    grid = (TOK // TILE,)
    tok_spec = pl.BlockSpec((TILE, D), lambda i: (i, 0))
    out_spec = pl.BlockSpec((TILE, TOP_K), lambda i: (i, 0))

    def full(a):
        return pl.BlockSpec(a.shape, lambda i: (0,) * a.ndim)

    ops = [Wv, bv.reshape(1, -1), Wo, bo.reshape(1, -1),
           nW1, nb1.reshape(1, -1), nW2, nb2.reshape(1, -1),
           nW3, nb3.reshape(1, -1),
           gW1, gb1.reshape(1, -1), gW2, gb2.reshape(1, -1),
           gW3, gb3.reshape(1, -1)]

    idx, w = pl.pallas_call(
        _router_kernel,
        grid=grid,
        in_specs=[tok_spec] + [full(o) for o in ops],
        out_specs=[out_spec, out_spec],
        out_shape=[jax.ShapeDtypeStruct((TOK, TOP_K), jnp.int32),
                   jax.ShapeDtypeStruct((TOK, TOP_K), jnp.float32)],
        scratch_shapes=[pltpu.VMEM((TILE, 64), jnp.float32),
                        pltpu.VMEM((TILE, 64), jnp.float32),
                        pltpu.VMEM((TILE, 256), jnp.float32)],
    )(x2, *ops)
    return idx, w


# speedup vs baseline: 1.2902x; 1.0209x over previous
import jax
import jax.numpy as jnp
from jax.experimental import pallas as pl
from jax.experimental.pallas import tpu as pltpu

D = 768
E = 64
TOP_K = 8
TOK = 32768
TILE = 2048


def _dot768(a, b_ref, acc_ref):
    acc_ref[...] = jnp.dot(a[:, 0:256], b_ref[0:256, :],
                           preferred_element_type=jnp.float32)
    acc_ref[...] = acc_ref[...] + jnp.dot(a[:, 256:512], b_ref[256:512, :],
                                          preferred_element_type=jnp.float32)
    acc_ref[...] = acc_ref[...] + jnp.dot(a[:, 512:768], b_ref[512:768, :],
                                          preferred_element_type=jnp.float32)
    return acc_ref[...]


def _router_kernel(x_ref, Wv_ref, bv_ref, Wo_ref, bo_ref,
                   nW1_ref, nb1_ref, nW2_ref, nb2_ref, nW3_ref, nb3_ref,
                   gW1_ref, gb1_ref, gW2_ref, gb2_ref, gW3_ref, gb3_ref,
                   idx_ref, w_ref, acc1_ref, acc2_ref, acc3_ref):
    xt = x_ref[...]
    ctx = _dot768(xt, Wv_ref, acc1_ref) + bv_ref[...]
    xs = jnp.dot(ctx, Wo_ref[...],
                 preferred_element_type=jnp.float32) + bo_ref[...]
    n = jax.nn.relu(_dot768(xs, nW1_ref, acc2_ref) + nb1_ref[...])
    n = jax.nn.relu(jnp.dot(n, nW2_ref[...],
                            preferred_element_type=jnp.float32) + nb2_ref[...])
    noisy = jax.nn.softplus(jnp.dot(n, nW3_ref[...],
                                    preferred_element_type=jnp.float32)
                            + nb3_ref[...])
    g = jax.nn.relu(_dot768(xs, gW1_ref, acc3_ref) + gb1_ref[...])
    g = jax.nn.relu(jnp.dot(g, gW2_ref[...],
                            preferred_element_type=jnp.float32) + gb2_ref[...])
    gating = jnp.dot(g, gW3_ref[...],
                     preferred_element_type=jnp.float32) + gb3_ref[...]
    mean = jnp.mean(noisy, axis=1, keepdims=True)
    var = jnp.sum((noisy - mean) ** 2, axis=1, keepdims=True) / (E - 1)
    std = jnp.sqrt(var)
    combined = gating + (noisy - mean) / (std + 1e-8)

    lanes = jax.lax.broadcasted_iota(jnp.int32, (TILE, E), 1).astype(jnp.float32)
    vals = combined
    top_v = []
    top_i = []
    for _ in range(TOP_K):
        m = jnp.max(vals, axis=1, keepdims=True)
        am = jnp.min(jnp.where(vals == m, lanes, jnp.float32(E)), axis=1,
                     keepdims=True)
        top_v.append(m)
        top_i.append(am)
        vals = jnp.where(lanes == am, -jnp.inf, vals)
    tv = jnp.concatenate(top_v, axis=1)
    ti = jnp.concatenate(top_i, axis=1)
    e = jnp.exp(tv - tv[:, 0:1])
    w = e / jnp.sum(e, axis=1, keepdims=True)
    idx_ref[...] = ti.astype(jnp.int32)
    w_ref[...] = w


@jax.jit
def kernel(x, Wq, bq, Wk, bk, Wv, bv, Wo, bo, gW1, gb1, gW2, gb2, gW3, gb3,
           nW1, nb1, nW2, nb2, nW3, nb3):
    x2 = x.reshape(TOK, D)
    grid = (TOK // TILE,)
    tok_spec = pl.BlockSpec((TILE, D), lambda i: (i, 0))
    out_spec = pl.BlockSpec((TILE, TOP_K), lambda i: (i, 0))

    def full(a):
        return pl.BlockSpec(a.shape, lambda i: (0,) * a.ndim)

    ops = [Wv, bv.reshape(1, -1), Wo, bo.reshape(1, -1),
           nW1, nb1.reshape(1, -1), nW2, nb2.reshape(1, -1),
           nW3, nb3.reshape(1, -1),
           gW1, gb1.reshape(1, -1), gW2, gb2.reshape(1, -1),
           gW3, gb3.reshape(1, -1)]

    idx, w = pl.pallas_call(
        _router_kernel,
        grid=grid,
        in_specs=[tok_spec] + [full(o) for o in ops],
        out_specs=[out_spec, out_spec],
        out_shape=[jax.ShapeDtypeStruct((TOK, TOP_K), jnp.int32),
                   jax.ShapeDtypeStruct((TOK, TOP_K), jnp.float32)],
        scratch_shapes=[pltpu.VMEM((TILE, 64), jnp.float32),
                        pltpu.VMEM((TILE, 64), jnp.float32),
                        pltpu.VMEM((TILE, 256), jnp.float32)],
    )(x2, *ops)
    return idx, w


# speedup vs baseline: 1.5033x; 1.1652x over previous
import jax
import jax.numpy as jnp
from jax.experimental import pallas as pl
from jax.experimental.pallas import tpu as pltpu

D = 768
E = 64
TOP_K = 8
TOK = 32768
TILE = 4096


def _dot768(a, b_ref, acc_ref):
    acc_ref[...] = jnp.dot(a[:, 0:256], b_ref[0:256, :],
                           preferred_element_type=jnp.float32)
    acc_ref[...] = acc_ref[...] + jnp.dot(a[:, 256:512], b_ref[256:512, :],
                                          preferred_element_type=jnp.float32)
    acc_ref[...] = acc_ref[...] + jnp.dot(a[:, 512:768], b_ref[512:768, :],
                                          preferred_element_type=jnp.float32)
    return acc_ref[...]


def _router_kernel(x_ref, Wv_ref, bv_ref, Wo_ref, bo_ref,
                   nW1_ref, nb1_ref, nW2_ref, nb2_ref, nW3_ref, nb3_ref,
                   gW1_ref, gb1_ref, gW2_ref, gb2_ref, gW3_ref, gb3_ref,
                   idx_ref, w_ref, acc1_ref, acc2_ref, acc3_ref):
    xt = x_ref[...]
    ctx = _dot768(xt, Wv_ref, acc1_ref) + bv_ref[...]
    xs = jnp.dot(ctx, Wo_ref[...],
                 preferred_element_type=jnp.float32) + bo_ref[...]
    n = jax.nn.relu(_dot768(xs, nW1_ref, acc2_ref) + nb1_ref[...])
    n = jax.nn.relu(jnp.dot(n, nW2_ref[...],
                            preferred_element_type=jnp.float32) + nb2_ref[...])
    noisy = jax.nn.softplus(jnp.dot(n, nW3_ref[...],
                                    preferred_element_type=jnp.float32)
                            + nb3_ref[...])
    g = jax.nn.relu(_dot768(xs, gW1_ref, acc3_ref) + gb1_ref[...])
    g = jax.nn.relu(jnp.dot(g, gW2_ref[...],
                            preferred_element_type=jnp.float32) + gb2_ref[...])
    gating = jnp.dot(g, gW3_ref[...],
                     preferred_element_type=jnp.float32) + gb3_ref[...]
    mean = jnp.mean(noisy, axis=1, keepdims=True)
    var = jnp.sum((noisy - mean) ** 2, axis=1, keepdims=True) / (E - 1)
    std = jnp.sqrt(var)
    combined = gating + (noisy - mean) / (std + 1e-8)

    lanes = jax.lax.broadcasted_iota(jnp.int32, (TILE, E), 1).astype(jnp.float32)
    vals = combined
    top_v = []
    top_i = []
    for _ in range(TOP_K):
        m = jnp.max(vals, axis=1, keepdims=True)
        am = jnp.min(jnp.where(vals == m, lanes, jnp.float32(E)), axis=1,
                     keepdims=True)
        top_v.append(m)
        top_i.append(am)
        vals = jnp.where(lanes == am, -jnp.inf, vals)
    tv = jnp.concatenate(top_v, axis=1)
    ti = jnp.concatenate(top_i, axis=1)
    e = jnp.exp(tv - tv[:, 0:1])
    w = e / jnp.sum(e, axis=1, keepdims=True)
    idx_ref[...] = ti.astype(jnp.int32)
    w_ref[...] = w


@jax.jit
def kernel(x, Wq, bq, Wk, bk, Wv, bv, Wo, bo, gW1, gb1, gW2, gb2, gW3, gb3,
           nW1, nb1, nW2, nb2, nW3, nb3):
    x2 = x.reshape(TOK, D)
    grid = (TOK // TILE,)
    tok_spec = pl.BlockSpec((TILE, D), lambda i: (i, 0))
    out_spec = pl.BlockSpec((TILE, TOP_K), lambda i: (i, 0))

    def full(a):
        return pl.BlockSpec(a.shape, lambda i: (0,) * a.ndim)

    ops = [Wv, bv.reshape(1, -1), Wo, bo.reshape(1, -1),
           nW1, nb1.reshape(1, -1), nW2, nb2.reshape(1, -1),
           nW3, nb3.reshape(1, -1),
           gW1, gb1.reshape(1, -1), gW2, gb2.reshape(1, -1),
           gW3, gb3.reshape(1, -1)]

    idx, w = pl.pallas_call(
        _router_kernel,
        grid=grid,
        in_specs=[tok_spec] + [full(o) for o in ops],
        out_specs=[out_spec, out_spec],
        out_shape=[jax.ShapeDtypeStruct((TOK, TOP_K), jnp.int32),
                   jax.ShapeDtypeStruct((TOK, TOP_K), jnp.float32)],
        scratch_shapes=[pltpu.VMEM((TILE, 64), jnp.float32),
                        pltpu.VMEM((TILE, 64), jnp.float32),
                        pltpu.VMEM((TILE, 256), jnp.float32)],
    )(x2, *ops)
    return idx, w
